# baseline (speedup 1.0000x reference)
import jax
import jax.numpy as jnp
from jax import lax
from jax.experimental import pallas as pl
from jax.experimental.pallas import tpu as pltpu
from jax.experimental.pallas import tpu_sc as plsc

N_NODES = 10000
N_EDGES = 320000
D_FEAT = 128
D_HID = 128
D_STRUCT_OUT = 64
NUM_CLASSES = 40

NC = 2
NS = 16
NW = NC * NS

CHUNK = 128
IDX_BLK = 40
N_IDX_BLKS = 2
CH_PER_TILE = IDX_BLK * N_IDX_BLKS
EDGES_PER_TILE = CH_PER_TILE * CHUNK
E_PAD = EDGES_PER_TILE * NW

N_PAD = N_NODES + 240
ROWS_PER_TILE = N_PAD // NS

NID_CH = -(-N_NODES // (NW * CHUNK))
NID_PAD = NID_CH * CHUNK * NW

ROW_BLK = 400
N_BLKS = N_NODES // ROW_BLK

_MESH = plsc.VectorSubcoreMesh(
    core_axis_name="c", subcore_axis_name="s", num_cores=NC, num_subcores=NS)


def _sc_agg_body(x, src_r, dst_r, zacc,
                 p0, p1,
                 idx_s_v, idx_d_v, rows_a, rows_b, acc_sh, sem_a, sem_b):
    c = lax.axis_index("c")
    s = lax.axis_index("s")
    t = c * NS + s
    row0 = pl.multiple_of(s * ROWS_PER_TILE, ROWS_PER_TILE)
    dsr = pl.ds(row0, ROWS_PER_TILE)

    pltpu.sync_copy(zacc, acc_sh.at[dsr])
    plsc.subcore_barrier()

    def ga(j, buf, sem):
        pltpu.async_copy(x.at[idx_s_v.at[j]], buf, sem)

    def wait(buf, sem):
        pltpu.make_async_copy(x.at[idx_s_v.at[0]], buf, sem).wait()

    def sca(j, buf):
        pltpu.sync_copy(buf, acc_sh.at[idx_d_v.at[j]], add=True)

    for h in range(N_IDX_BLKS):
        pltpu.sync_copy(src_r.at[t, h], idx_s_v)
        pltpu.sync_copy(dst_r.at[t, h], idx_d_v)

        def body(j, carry):
            ga(j, rows_a, sem_a)
            wait(rows_a, sem_a)
            sca(j, rows_a)
            return carry

        lax.fori_loop(0, IDX_BLK, body, 0)

    plsc.subcore_barrier()

    @pl.when(c == 0)
    def _():
        pltpu.sync_copy(acc_sh.at[dsr], p0.at[dsr])

    @pl.when(c == 1)
    def _():
        pltpu.sync_copy(acc_sh.at[dsr], p1.at[dsr])


_sc_aggregate = pl.kernel(
    _sc_agg_body,
    out_type=(
        jax.ShapeDtypeStruct((N_PAD, D_FEAT), jnp.float32),
        jax.ShapeDtypeStruct((N_PAD, D_FEAT), jnp.float32),
    ),
    mesh=_MESH,
    scratch_types=[
        pltpu.VMEM((IDX_BLK, CHUNK), jnp.int32),
        pltpu.VMEM((IDX_BLK, CHUNK), jnp.int32),
        pltpu.VMEM((CHUNK, D_FEAT), jnp.float32),
        pltpu.VMEM((CHUNK, D_FEAT), jnp.float32),
        pltpu.VMEM_SHARED((N_PAD, D_FEAT), jnp.float32),
        pltpu.SemaphoreType.DMA,
        pltpu.SemaphoreType.DMA,
    ],
)


def _sc_count_body(dst_r, zcnt,
                   c0, c1,
                   idx_d_v, cnt_v, red_v, res_v, cnt_sh):
    c = lax.axis_index("c")
    s = lax.axis_index("s")
    t = c * NS + s
    row0 = pl.multiple_of(s * ROWS_PER_TILE, ROWS_PER_TILE)
    dsr = pl.ds(row0, ROWS_PER_TILE)

    pltpu.sync_copy(dst_r.at[t], idx_d_v)
    pltpu.sync_copy(zcnt, cnt_v)
    ones16 = jnp.ones((16,), jnp.float32)

    for h in range(N_IDX_BLKS):
        def step(j, carry):
            def sub(k, carry2):
                idx = idx_d_v[h, j, pl.ds(k * 16, 16)]
                plsc.addupdate_scatter(cnt_v, [idx], ones16)
                return carry2
            return lax.fori_loop(0, CHUNK // 16, sub, carry)

        lax.fori_loop(0, IDX_BLK, step, 0)

    pltpu.sync_copy(cnt_v, cnt_sh.at[s])
    plsc.subcore_barrier()
    pltpu.sync_copy(cnt_sh.at[:, dsr], red_v)

    def red(r, carry):
        base = pl.multiple_of(r * 16, 16)
        acc = red_v[0, pl.ds(base, 16)]
        for q in range(1, NS):
            acc = acc + red_v[q, pl.ds(base, 16)]
        res_v[pl.ds(base, 16)] = acc
        return carry

    lax.fori_loop(0, ROWS_PER_TILE // 16, red, 0)

    @pl.when(c == 0)
    def _():
        pltpu.sync_copy(res_v, c0.at[dsr])

    @pl.when(c == 1)
    def _():
        pltpu.sync_copy(res_v, c1.at[dsr])


_sc_count = pl.kernel(
    _sc_count_body,
    out_type=(
        jax.ShapeDtypeStruct((N_PAD,), jnp.float32),
        jax.ShapeDtypeStruct((N_PAD,), jnp.float32),
    ),
    mesh=_MESH,
    compiler_params=pltpu.CompilerParams(needs_layout_passes=False),
    scratch_types=[
        pltpu.VMEM((N_IDX_BLKS, IDX_BLK, CHUNK), jnp.int32),
        pltpu.VMEM((N_PAD,), jnp.float32),
        pltpu.VMEM((NS, ROWS_PER_TILE), jnp.float32),
        pltpu.VMEM((ROWS_PER_TILE,), jnp.float32),
        pltpu.VMEM_SHARED((NS, N_PAD), jnp.float32),
    ],
)


def _sc_gather_body(t0, t1, tw, nid_r,
                    g0, g1, gi,
                    nid_v, rows_v, sem):
    c = lax.axis_index("c")
    s = lax.axis_index("s")
    t = c * NS + s
    pltpu.sync_copy(nid_r.at[t], nid_v)

    def step(j, carry):
        base = pl.multiple_of(t * (NID_CH * CHUNK) + j * CHUNK, CHUNK)
        dso = pl.ds(base, CHUNK)
        pltpu.async_copy(t0.at[nid_v.at[j]], rows_v, sem).wait()
        pltpu.sync_copy(rows_v, g0.at[dso])
        pltpu.async_copy(t1.at[nid_v.at[j]], rows_v, sem).wait()
        pltpu.sync_copy(rows_v, g1.at[dso])
        pltpu.async_copy(tw.at[nid_v.at[j]], rows_v, sem).wait()
        pltpu.sync_copy(rows_v, gi.at[dso])
        return carry

    lax.fori_loop(0, NID_CH, step, 0)


_sc_gather = pl.kernel(
    _sc_gather_body,
    out_type=(
        jax.ShapeDtypeStruct((NID_PAD, D_FEAT), jnp.float32),
        jax.ShapeDtypeStruct((NID_PAD, D_FEAT), jnp.float32),
        jax.ShapeDtypeStruct((NID_PAD, D_FEAT), jnp.float32),
    ),
    mesh=_MESH,
    scratch_types=[
        pltpu.VMEM((NID_CH, CHUNK), jnp.int32),
        pltpu.VMEM((CHUNK, D_FEAT), jnp.float32),
        pltpu.SemaphoreType.DMA,
    ],
)


def _tc_layer1_body(p0, p1, c0, c1, w, b, s_out, iw_out):
    cnt = c0[...] + c1[...]
    inv = 1.0 / jnp.maximum(cnt, 1.0)
    agg = (p0[...] + p1[...]) * inv
    h = jnp.dot(agg, w[...], preferred_element_type=jnp.float32) + b[...]
    s_out[...] = jnp.maximum(h, 0.0)
    iw_out[...] = jnp.broadcast_to(inv, (ROW_BLK, D_FEAT))


def _tc_layer1(p0, p1, c0, c1, w, b):
    return pl.pallas_call(
        _tc_layer1_body,
        grid=(N_BLKS,),
        in_specs=[
            pl.BlockSpec((ROW_BLK, D_FEAT), lambda i: (i, 0)),
            pl.BlockSpec((ROW_BLK, D_FEAT), lambda i: (i, 0)),
            pl.BlockSpec((ROW_BLK, 1), lambda i: (i, 0)),
            pl.BlockSpec((ROW_BLK, 1), lambda i: (i, 0)),
            pl.BlockSpec((D_FEAT, D_HID), lambda i: (0, 0)),
            pl.BlockSpec((1, D_HID), lambda i: (0, 0)),
        ],
        out_specs=[
            pl.BlockSpec((ROW_BLK, D_HID), lambda i: (i, 0)),
            pl.BlockSpec((ROW_BLK, D_FEAT), lambda i: (i, 0)),
        ],
        out_shape=[
            jax.ShapeDtypeStruct((N_NODES, D_HID), jnp.float32),
            jax.ShapeDtypeStruct((N_NODES, D_FEAT), jnp.float32),
        ],
    )(p0, p1, c0, c1, w, b.reshape(1, D_HID))


def _tc_final_body(p0, p1, c0, c1, g0, g1, gi, cx,
                   wc, bc, wg2, bg2, wl, bl, s_out, cls_out):
    cnt = c0[...] + c1[...]
    inv = 1.0 / jnp.maximum(cnt, 1.0)
    agg = (p0[...] + p1[...]) * inv
    S = jnp.dot(agg, wg2[...], preferred_element_type=jnp.float32) + bg2[...]
    s_out[...] = S

    aggg = (g0[...] + g1[...]) * gi[:, 0:1]
    xs = jnp.dot(aggg, wg2[...], preferred_element_type=jnp.float32) + bg2[...]

    wl_top = wl[0:D_STRUCT_OUT, :]
    wl_bot = wl[D_STRUCT_OUT:, :]
    wcl = jnp.dot(wc[...], wl_top, preferred_element_type=jnp.float32)
    bfold = jnp.dot(bc[...], wl_top, preferred_element_type=jnp.float32) + bl[...]
    logits = (jnp.dot(cx[...], wcl, preferred_element_type=jnp.float32)
              + jnp.dot(xs, wl_bot, preferred_element_type=jnp.float32)
              + bfold)
    m = jnp.max(logits, axis=1, keepdims=True)
    e = jnp.exp(logits - m)
    cls_out[...] = e / jnp.sum(e, axis=1, keepdims=True)


def _tc_final(p0, p1, c0, c1, g0, g1, gi, cx, wc, bc, wg2, bg2, wl, bl):
    full = lambda r, k: pl.BlockSpec((r, k), lambda i: (0, 0))
    blk = lambda k: pl.BlockSpec((ROW_BLK, k), lambda i: (i, 0))
    return pl.pallas_call(
        _tc_final_body,
        grid=(N_BLKS,),
        in_specs=[
            blk(D_FEAT), blk(D_FEAT), blk(1), blk(1),
            blk(D_FEAT), blk(D_FEAT), blk(D_FEAT),
            blk(D_FEAT),
            full(D_FEAT, D_STRUCT_OUT), full(1, D_STRUCT_OUT),
            full(D_HID, D_STRUCT_OUT), full(1, D_STRUCT_OUT),
            full(D_FEAT, NUM_CLASSES), full(1, NUM_CLASSES),
        ],
        out_specs=[
            pl.BlockSpec((ROW_BLK, D_STRUCT_OUT), lambda i: (i, 0)),
            pl.BlockSpec((ROW_BLK, NUM_CLASSES), lambda i: (i, 0)),
        ],
        out_shape=[
            jax.ShapeDtypeStruct((N_NODES, D_STRUCT_OUT), jnp.float32),
            jax.ShapeDtypeStruct((N_NODES, NUM_CLASSES), jnp.float32),
        ],
    )(p0, p1, c0, c1, g0, g1, gi, cx,
      wc, bc.reshape(1, -1), wg2, bg2.reshape(1, -1), wl, bl.reshape(1, -1))


def kernel(client_x, structural_features, node_ids, edge_index,
           W_c, b_c, W_g1, b_g1, W_g2, b_g2, W_l, b_l):
    src = edge_index[0]
    dst = edge_index[1]
    epad = E_PAD - N_EDGES
    src_r = jnp.concatenate(
        [src, jnp.zeros((epad,), jnp.int32)]
    ).reshape(NW, N_IDX_BLKS, IDX_BLK, CHUNK)
    dst_r = jnp.concatenate(
        [dst, jnp.full((epad,), N_NODES, jnp.int32)]
    ).reshape(NW, N_IDX_BLKS, IDX_BLK, CHUNK)
    nid_r = jnp.concatenate(
        [node_ids, jnp.zeros((NID_PAD - N_NODES,), jnp.int32)]
    ).reshape(NW, NID_CH, CHUNK)
    zacc = jnp.zeros((ROWS_PER_TILE, D_FEAT), jnp.float32)
    zcnt = jnp.zeros((N_PAD,), jnp.float32)

    c0, c1 = _sc_count(dst_r, zcnt)
    c0 = c0.reshape(N_PAD, 1)
    c1 = c1.reshape(N_PAD, 1)
    p0, p1 = _sc_aggregate(structural_features, src_r, dst_r, zacc)
    s, iw = _tc_layer1(p0, p1, c0, c1, W_g1, b_g1)
    q0, q1 = _sc_aggregate(s, src_r, dst_r, zacc)
    g0, g1, gi = _sc_gather(q0, q1, iw, nid_r)
    S, out_client = _tc_final(
        q0, q1, c0, c1, g0, g1, gi, client_x,
        W_c, b_c, W_g2, b_g2, W_l, b_l)
    return (S, out_client)


# speedup vs baseline: 1.0037x; 1.0037x over previous
import jax
import jax.numpy as jnp
from jax import lax
from jax.experimental import pallas as pl
from jax.experimental.pallas import tpu as pltpu
from jax.experimental.pallas import tpu_sc as plsc

N_NODES = 10000
N_EDGES = 320000
D_FEAT = 128
D_HID = 128
D_STRUCT_OUT = 64
NUM_CLASSES = 40

NC = 2
NS = 16
NW = NC * NS

CHUNK = 128
IDX_BLK = 40
N_IDX_BLKS = 2
CH_PER_TILE = IDX_BLK * N_IDX_BLKS
EDGES_PER_TILE = CH_PER_TILE * CHUNK
E_PAD = EDGES_PER_TILE * NW

N_PAD = N_NODES + 240
ROWS_PER_TILE = N_PAD // NS

NID_CH = -(-N_NODES // (NW * CHUNK))
NID_PAD = NID_CH * CHUNK * NW

ROW_BLK = 400
N_BLKS = N_NODES // ROW_BLK

_MESH = plsc.VectorSubcoreMesh(
    core_axis_name="c", subcore_axis_name="s", num_cores=NC, num_subcores=NS)


def _sc_agg_body(x, src_r, dst_r, zacc,
                 p0, p1,
                 idx_s_v, idx_d_v, rows_a, rows_b, acc_sh, sem_a, sem_b):
    c = lax.axis_index("c")
    s = lax.axis_index("s")
    t = c * NS + s
    row0 = pl.multiple_of(s * ROWS_PER_TILE, ROWS_PER_TILE)
    dsr = pl.ds(row0, ROWS_PER_TILE)

    pltpu.sync_copy(zacc, acc_sh.at[dsr])
    plsc.subcore_barrier()

    pltpu.sync_copy(src_r.at[t], idx_s_v)
    pltpu.sync_copy(dst_r.at[t], idx_d_v)

    def body(j, carry):
        pltpu.async_copy(x.at[idx_s_v.at[j]], rows_a, sem_a).wait()
        pltpu.sync_copy(rows_a, acc_sh.at[idx_d_v.at[j]], add=True)
        return carry

    lax.fori_loop(0, CH_PER_TILE, body, 0)

    plsc.subcore_barrier()

    @pl.when(c == 0)
    def _():
        pltpu.sync_copy(acc_sh.at[dsr], p0.at[dsr])

    @pl.when(c == 1)
    def _():
        pltpu.sync_copy(acc_sh.at[dsr], p1.at[dsr])


_sc_aggregate = pl.kernel(
    _sc_agg_body,
    out_type=(
        jax.ShapeDtypeStruct((N_PAD, D_FEAT), jnp.float32),
        jax.ShapeDtypeStruct((N_PAD, D_FEAT), jnp.float32),
    ),
    mesh=_MESH,
    scratch_types=[
        pltpu.VMEM((CH_PER_TILE, CHUNK), jnp.int32),
        pltpu.VMEM((CH_PER_TILE, CHUNK), jnp.int32),
        pltpu.VMEM((CHUNK, D_FEAT), jnp.float32),
        pltpu.VMEM((CHUNK, D_FEAT), jnp.float32),
        pltpu.VMEM_SHARED((N_PAD, D_FEAT), jnp.float32),
        pltpu.SemaphoreType.DMA,
        pltpu.SemaphoreType.DMA,
    ],
)


def _sc_count_body(dst_r, zcnt,
                   c0, c1,
                   idx_d_v, cnt_v, red_v, res_v, cnt_sh):
    c = lax.axis_index("c")
    s = lax.axis_index("s")
    t = c * NS + s
    row0 = pl.multiple_of(s * ROWS_PER_TILE, ROWS_PER_TILE)
    dsr = pl.ds(row0, ROWS_PER_TILE)

    pltpu.sync_copy(dst_r.at[t], idx_d_v)
    pltpu.sync_copy(zcnt, cnt_v)
    ones16 = jnp.ones((16,), jnp.float32)

    def step(j, carry):
        def sub(k, carry2):
            idx = idx_d_v[j, pl.ds(k * 16, 16)]
            plsc.addupdate_scatter(cnt_v, [idx], ones16)
            return carry2
        return lax.fori_loop(0, CHUNK // 16, sub, carry)

    lax.fori_loop(0, CH_PER_TILE, step, 0)

    pltpu.sync_copy(cnt_v, cnt_sh.at[s])
    plsc.subcore_barrier()
    pltpu.sync_copy(cnt_sh.at[:, dsr], red_v)

    def red(r, carry):
        base = pl.multiple_of(r * 16, 16)
        acc = red_v[0, pl.ds(base, 16)]
        for q in range(1, NS):
            acc = acc + red_v[q, pl.ds(base, 16)]
        res_v[pl.ds(base, 16)] = acc
        return carry

    lax.fori_loop(0, ROWS_PER_TILE // 16, red, 0)

    @pl.when(c == 0)
    def _():
        pltpu.sync_copy(res_v, c0.at[dsr])

    @pl.when(c == 1)
    def _():
        pltpu.sync_copy(res_v, c1.at[dsr])


_sc_count = pl.kernel(
    _sc_count_body,
    out_type=(
        jax.ShapeDtypeStruct((N_PAD,), jnp.float32),
        jax.ShapeDtypeStruct((N_PAD,), jnp.float32),
    ),
    mesh=_MESH,
    compiler_params=pltpu.CompilerParams(needs_layout_passes=False),
    scratch_types=[
        pltpu.VMEM((CH_PER_TILE, CHUNK), jnp.int32),
        pltpu.VMEM((N_PAD,), jnp.float32),
        pltpu.VMEM((NS, ROWS_PER_TILE), jnp.float32),
        pltpu.VMEM((ROWS_PER_TILE,), jnp.float32),
        pltpu.VMEM_SHARED((NS, N_PAD), jnp.float32),
    ],
)


def _sc_gather_body(t0, t1, tw, nid_r,
                    g0, g1, gi,
                    nid_v, rows_v, sem):
    c = lax.axis_index("c")
    s = lax.axis_index("s")
    t = c * NS + s
    pltpu.sync_copy(nid_r.at[t], nid_v)

    def step(j, carry):
        base = pl.multiple_of(t * (NID_CH * CHUNK) + j * CHUNK, CHUNK)
        dso = pl.ds(base, CHUNK)
        pltpu.async_copy(t0.at[nid_v.at[j]], rows_v, sem).wait()
        pltpu.sync_copy(rows_v, g0.at[dso])
        pltpu.async_copy(t1.at[nid_v.at[j]], rows_v, sem).wait()
        pltpu.sync_copy(rows_v, g1.at[dso])
        pltpu.async_copy(tw.at[nid_v.at[j]], rows_v, sem).wait()
        pltpu.sync_copy(rows_v, gi.at[dso])
        return carry

    lax.fori_loop(0, NID_CH, step, 0)


_sc_gather = pl.kernel(
    _sc_gather_body,
    out_type=(
        jax.ShapeDtypeStruct((NID_PAD, D_FEAT), jnp.float32),
        jax.ShapeDtypeStruct((NID_PAD, D_FEAT), jnp.float32),
        jax.ShapeDtypeStruct((NID_PAD, D_FEAT), jnp.float32),
    ),
    mesh=_MESH,
    scratch_types=[
        pltpu.VMEM((NID_CH, CHUNK), jnp.int32),
        pltpu.VMEM((CHUNK, D_FEAT), jnp.float32),
        pltpu.SemaphoreType.DMA,
    ],
)


def _tc_layer1_body(p0, p1, c0, c1, w, b, s_out, iw_out):
    cnt = c0[...] + c1[...]
    inv = 1.0 / jnp.maximum(cnt, 1.0)
    agg = (p0[...] + p1[...]) * inv
    h = jnp.dot(agg, w[...], preferred_element_type=jnp.float32) + b[...]
    s_out[...] = jnp.maximum(h, 0.0)
    iw_out[...] = jnp.broadcast_to(inv, (ROW_BLK, D_FEAT))


def _tc_layer1(p0, p1, c0, c1, w, b):
    return pl.pallas_call(
        _tc_layer1_body,
        grid=(N_BLKS,),
        in_specs=[
            pl.BlockSpec((ROW_BLK, D_FEAT), lambda i: (i, 0)),
            pl.BlockSpec((ROW_BLK, D_FEAT), lambda i: (i, 0)),
            pl.BlockSpec((ROW_BLK, 1), lambda i: (i, 0)),
            pl.BlockSpec((ROW_BLK, 1), lambda i: (i, 0)),
            pl.BlockSpec((D_FEAT, D_HID), lambda i: (0, 0)),
            pl.BlockSpec((1, D_HID), lambda i: (0, 0)),
        ],
        out_specs=[
            pl.BlockSpec((ROW_BLK, D_HID), lambda i: (i, 0)),
            pl.BlockSpec((ROW_BLK, D_FEAT), lambda i: (i, 0)),
        ],
        out_shape=[
            jax.ShapeDtypeStruct((N_NODES, D_HID), jnp.float32),
            jax.ShapeDtypeStruct((N_NODES, D_FEAT), jnp.float32),
        ],
    )(p0, p1, c0, c1, w, b.reshape(1, D_HID))


def _tc_final_body(p0, p1, c0, c1, g0, g1, gi, cx,
                   wc, bc, wg2, bg2, wl, bl, s_out, cls_out):
    cnt = c0[...] + c1[...]
    inv = 1.0 / jnp.maximum(cnt, 1.0)
    agg = (p0[...] + p1[...]) * inv
    S = jnp.dot(agg, wg2[...], preferred_element_type=jnp.float32) + bg2[...]
    s_out[...] = S

    aggg = (g0[...] + g1[...]) * gi[:, 0:1]
    xs = jnp.dot(aggg, wg2[...], preferred_element_type=jnp.float32) + bg2[...]

    wl_top = wl[0:D_STRUCT_OUT, :]
    wl_bot = wl[D_STRUCT_OUT:, :]
    wcl = jnp.dot(wc[...], wl_top, preferred_element_type=jnp.float32)
    bfold = jnp.dot(bc[...], wl_top, preferred_element_type=jnp.float32) + bl[...]
    logits = (jnp.dot(cx[...], wcl, preferred_element_type=jnp.float32)
              + jnp.dot(xs, wl_bot, preferred_element_type=jnp.float32)
              + bfold)
    m = jnp.max(logits, axis=1, keepdims=True)
    e = jnp.exp(logits - m)
    cls_out[...] = e / jnp.sum(e, axis=1, keepdims=True)


def _tc_final(p0, p1, c0, c1, g0, g1, gi, cx, wc, bc, wg2, bg2, wl, bl):
    full = lambda r, k: pl.BlockSpec((r, k), lambda i: (0, 0))
    blk = lambda k: pl.BlockSpec((ROW_BLK, k), lambda i: (i, 0))
    return pl.pallas_call(
        _tc_final_body,
        grid=(N_BLKS,),
        in_specs=[
            blk(D_FEAT), blk(D_FEAT), blk(1), blk(1),
            blk(D_FEAT), blk(D_FEAT), blk(D_FEAT),
            blk(D_FEAT),
            full(D_FEAT, D_STRUCT_OUT), full(1, D_STRUCT_OUT),
            full(D_HID, D_STRUCT_OUT), full(1, D_STRUCT_OUT),
            full(D_FEAT, NUM_CLASSES), full(1, NUM_CLASSES),
        ],
        out_specs=[
            pl.BlockSpec((ROW_BLK, D_STRUCT_OUT), lambda i: (i, 0)),
            pl.BlockSpec((ROW_BLK, NUM_CLASSES), lambda i: (i, 0)),
        ],
        out_shape=[
            jax.ShapeDtypeStruct((N_NODES, D_STRUCT_OUT), jnp.float32),
            jax.ShapeDtypeStruct((N_NODES, NUM_CLASSES), jnp.float32),
        ],
    )(p0, p1, c0, c1, g0, g1, gi, cx,
      wc, bc.reshape(1, -1), wg2, bg2.reshape(1, -1), wl, bl.reshape(1, -1))


def kernel(client_x, structural_features, node_ids, edge_index,
           W_c, b_c, W_g1, b_g1, W_g2, b_g2, W_l, b_l):
    src = edge_index[0]
    dst = edge_index[1]
    epad = E_PAD - N_EDGES
    src_r = jnp.concatenate(
        [src, jnp.zeros((epad,), jnp.int32)]).reshape(NW, CH_PER_TILE, CHUNK)
    dst_r = jnp.concatenate(
        [dst, jnp.full((epad,), N_NODES, jnp.int32)]).reshape(NW, CH_PER_TILE, CHUNK)
    nid_r = jnp.concatenate(
        [node_ids, jnp.zeros((NID_PAD - N_NODES,), jnp.int32)]
    ).reshape(NW, NID_CH, CHUNK)
    zacc = jnp.zeros((ROWS_PER_TILE, D_FEAT), jnp.float32)
    zcnt = jnp.zeros((N_PAD,), jnp.float32)

    c0, c1 = _sc_count(dst_r, zcnt)
    c0 = c0.reshape(N_PAD, 1)
    c1 = c1.reshape(N_PAD, 1)
    p0, p1 = _sc_aggregate(structural_features, src_r, dst_r, zacc)
    s, iw = _tc_layer1(p0, p1, c0, c1, W_g1, b_g1)
    q0, q1 = _sc_aggregate(s, src_r, dst_r, zacc)
    g0, g1, gi = _sc_gather(q0, q1, iw, nid_r)
    S, out_client = _tc_final(
        q0, q1, c0, c1, g0, g1, gi, client_x,
        W_c, b_c, W_g2, b_g2, W_l, b_l)
    return (S, out_client)


# speedup vs baseline: 1.3701x; 1.3651x over previous
import jax
import jax.numpy as jnp
from jax import lax
from jax.experimental import pallas as pl
from jax.experimental.pallas import tpu as pltpu
from jax.experimental.pallas import tpu_sc as plsc

N_NODES = 10000
N_EDGES = 320000
D_FEAT = 128
D_HID = 128
D_STRUCT_OUT = 64
NUM_CLASSES = 40

NC = 2
NS = 16
NW = NC * NS

CHUNK = 128
CH_PER_TILE = -(-N_EDGES // (NW * CHUNK))
EDGES_PER_TILE = CH_PER_TILE * CHUNK
E_PAD = EDGES_PER_TILE * NW

N_PAD = N_NODES + 240
ROWS_PER_TILE = N_PAD // NS

NID_CH = -(-N_NODES // (NW * CHUNK))
NID_PAD = NID_CH * CHUNK * NW

ROW_BLK = 400
N_BLKS = N_NODES // ROW_BLK

_MESH = plsc.VectorSubcoreMesh(
    core_axis_name="c", subcore_axis_name="s", num_cores=NC, num_subcores=NS)


def _sc_agg_body(x, src_r, dst_r, zacc,
                 p0, p1,
                 idx_s_v, idx_d_v, rows_a, acc_sh, sem_a):
    c = lax.axis_index("c")
    s = lax.axis_index("s")
    t = c * NS + s
    row0 = pl.multiple_of(s * ROWS_PER_TILE, ROWS_PER_TILE)
    dsr = pl.ds(row0, ROWS_PER_TILE)

    pltpu.sync_copy(zacc, acc_sh.at[dsr])
    plsc.subcore_barrier()

    pltpu.sync_copy(src_r.at[t], idx_s_v)
    pltpu.sync_copy(dst_r.at[t], idx_d_v)

    def body(j, carry):
        pltpu.async_copy(x.at[idx_s_v.at[j]], rows_a, sem_a).wait()
        pltpu.sync_copy(rows_a, acc_sh.at[idx_d_v.at[j]], add=True)
        return carry

    lax.fori_loop(0, CH_PER_TILE, body, 0)

    plsc.subcore_barrier()

    @pl.when(c == 0)
    def _():
        pltpu.sync_copy(acc_sh.at[dsr], p0.at[dsr])

    @pl.when(c == 1)
    def _():
        pltpu.sync_copy(acc_sh.at[dsr], p1.at[dsr])


_sc_aggregate = pl.kernel(
    _sc_agg_body,
    out_type=(
        jax.ShapeDtypeStruct((N_PAD, D_FEAT), jnp.float32),
        jax.ShapeDtypeStruct((N_PAD, D_FEAT), jnp.float32),
    ),
    mesh=_MESH,
    scratch_types=[
        pltpu.VMEM((CH_PER_TILE, CHUNK), jnp.int32),
        pltpu.VMEM((CH_PER_TILE, CHUNK), jnp.int32),
        pltpu.VMEM((CHUNK, D_FEAT), jnp.float32),
        pltpu.VMEM_SHARED((N_PAD, D_FEAT), jnp.float32),
        pltpu.SemaphoreType.DMA,
    ],
)


def _sc_count_body(dst_r, zcnt,
                   c0, c1,
                   idx_d_v, cnt_v, red_v, res_v, cnt_sh):
    c = lax.axis_index("c")
    s = lax.axis_index("s")
    t = c * NS + s
    row0 = pl.multiple_of(s * ROWS_PER_TILE, ROWS_PER_TILE)
    dsr = pl.ds(row0, ROWS_PER_TILE)

    pltpu.sync_copy(dst_r.at[t], idx_d_v)
    pltpu.sync_copy(zcnt, cnt_v)
    ones16 = jnp.ones((16,), jnp.float32)

    def step(j, carry):
        def sub(k, carry2):
            idx = idx_d_v[j, pl.ds(k * 16, 16)]
            plsc.addupdate_scatter(cnt_v, [idx], ones16)
            return carry2
        return lax.fori_loop(0, CHUNK // 16, sub, carry)

    lax.fori_loop(0, CH_PER_TILE, step, 0)

    pltpu.sync_copy(cnt_v, cnt_sh.at[s])
    plsc.subcore_barrier()
    pltpu.sync_copy(cnt_sh.at[:, dsr], red_v)

    def red(r, carry):
        base = pl.multiple_of(r * 16, 16)
        acc = red_v[0, pl.ds(base, 16)]
        for q in range(1, NS):
            acc = acc + red_v[q, pl.ds(base, 16)]
        res_v[pl.ds(base, 16)] = acc
        return carry

    lax.fori_loop(0, ROWS_PER_TILE // 16, red, 0)

    @pl.when(c == 0)
    def _():
        pltpu.sync_copy(res_v, c0.at[dsr])

    @pl.when(c == 1)
    def _():
        pltpu.sync_copy(res_v, c1.at[dsr])


_sc_count = pl.kernel(
    _sc_count_body,
    out_type=(
        jax.ShapeDtypeStruct((N_PAD,), jnp.float32),
        jax.ShapeDtypeStruct((N_PAD,), jnp.float32),
    ),
    mesh=_MESH,
    compiler_params=pltpu.CompilerParams(needs_layout_passes=False),
    scratch_types=[
        pltpu.VMEM((CH_PER_TILE, CHUNK), jnp.int32),
        pltpu.VMEM((N_PAD,), jnp.float32),
        pltpu.VMEM((NS, ROWS_PER_TILE), jnp.float32),
        pltpu.VMEM((ROWS_PER_TILE,), jnp.float32),
        pltpu.VMEM_SHARED((NS, N_PAD), jnp.float32),
    ],
)


def _sc_gather_body(t0, t1, tw, nid_r,
                    g0, g1, gi,
                    nid_v, rows_v, sem):
    c = lax.axis_index("c")
    s = lax.axis_index("s")
    t = c * NS + s
    pltpu.sync_copy(nid_r.at[t], nid_v)

    def step(j, carry):
        base = pl.multiple_of(t * (NID_CH * CHUNK) + j * CHUNK, CHUNK)
        dso = pl.ds(base, CHUNK)
        pltpu.async_copy(t0.at[nid_v.at[j]], rows_v, sem).wait()
        pltpu.sync_copy(rows_v, g0.at[dso])
        pltpu.async_copy(t1.at[nid_v.at[j]], rows_v, sem).wait()
        pltpu.sync_copy(rows_v, g1.at[dso])
        pltpu.async_copy(tw.at[nid_v.at[j]], rows_v, sem).wait()
        pltpu.sync_copy(rows_v, gi.at[dso])
        return carry

    lax.fori_loop(0, NID_CH, step, 0)


_sc_gather = pl.kernel(
    _sc_gather_body,
    out_type=(
        jax.ShapeDtypeStruct((NID_PAD, D_FEAT), jnp.float32),
        jax.ShapeDtypeStruct((NID_PAD, D_FEAT), jnp.float32),
        jax.ShapeDtypeStruct((NID_PAD, D_FEAT), jnp.float32),
    ),
    mesh=_MESH,
    scratch_types=[
        pltpu.VMEM((NID_CH, CHUNK), jnp.int32),
        pltpu.VMEM((CHUNK, D_FEAT), jnp.float32),
        pltpu.SemaphoreType.DMA,
    ],
)


def _tc_layer1_body(p0, p1, c0, c1, w, b, s_out, iw_out):
    cnt = c0[...] + c1[...]
    inv = 1.0 / jnp.maximum(cnt, 1.0)
    agg = (p0[...] + p1[...]) * inv
    h = jnp.dot(agg, w[...], preferred_element_type=jnp.float32) + b[...]
    s_out[...] = jnp.maximum(h, 0.0)
    iw_out[...] = jnp.broadcast_to(inv, (ROW_BLK, D_FEAT))


def _tc_layer1(p0, p1, c0, c1, w, b):
    return pl.pallas_call(
        _tc_layer1_body,
        grid=(N_BLKS,),
        in_specs=[
            pl.BlockSpec((ROW_BLK, D_FEAT), lambda i: (i, 0)),
            pl.BlockSpec((ROW_BLK, D_FEAT), lambda i: (i, 0)),
            pl.BlockSpec((ROW_BLK, 1), lambda i: (i, 0)),
            pl.BlockSpec((ROW_BLK, 1), lambda i: (i, 0)),
            pl.BlockSpec((D_FEAT, D_HID), lambda i: (0, 0)),
            pl.BlockSpec((1, D_HID), lambda i: (0, 0)),
        ],
        out_specs=[
            pl.BlockSpec((ROW_BLK, D_HID), lambda i: (i, 0)),
            pl.BlockSpec((ROW_BLK, D_FEAT), lambda i: (i, 0)),
        ],
        out_shape=[
            jax.ShapeDtypeStruct((N_NODES, D_HID), jnp.float32),
            jax.ShapeDtypeStruct((N_NODES, D_FEAT), jnp.float32),
        ],
    )(p0, p1, c0, c1, w, b.reshape(1, D_HID))


def _tc_final_body(p0, p1, c0, c1, g0, g1, gi, cx,
                   wc, bc, wg2, bg2, wl, bl, s_out, cls_out):
    cnt = c0[...] + c1[...]
    inv = 1.0 / jnp.maximum(cnt, 1.0)
    agg = (p0[...] + p1[...]) * inv
    S = jnp.dot(agg, wg2[...], preferred_element_type=jnp.float32) + bg2[...]
    s_out[...] = S

    aggg = (g0[...] + g1[...]) * gi[:, 0:1]
    xs = jnp.dot(aggg, wg2[...], preferred_element_type=jnp.float32) + bg2[...]

    wl_top = wl[0:D_STRUCT_OUT, :]
    wl_bot = wl[D_STRUCT_OUT:, :]
    wcl = jnp.dot(wc[...], wl_top, preferred_element_type=jnp.float32)
    bfold = jnp.dot(bc[...], wl_top, preferred_element_type=jnp.float32) + bl[...]
    logits = (jnp.dot(cx[...], wcl, preferred_element_type=jnp.float32)
              + jnp.dot(xs, wl_bot, preferred_element_type=jnp.float32)
              + bfold)
    m = jnp.max(logits, axis=1, keepdims=True)
    e = jnp.exp(logits - m)
    cls_out[...] = e / jnp.sum(e, axis=1, keepdims=True)


def _tc_final(p0, p1, c0, c1, g0, g1, gi, cx, wc, bc, wg2, bg2, wl, bl):
    full = lambda r, k: pl.BlockSpec((r, k), lambda i: (0, 0))
    blk = lambda k: pl.BlockSpec((ROW_BLK, k), lambda i: (i, 0))
    return pl.pallas_call(
        _tc_final_body,
        grid=(N_BLKS,),
        in_specs=[
            blk(D_FEAT), blk(D_FEAT), blk(1), blk(1),
            blk(D_FEAT), blk(D_FEAT), blk(D_FEAT),
            blk(D_FEAT),
            full(D_FEAT, D_STRUCT_OUT), full(1, D_STRUCT_OUT),
            full(D_HID, D_STRUCT_OUT), full(1, D_STRUCT_OUT),
            full(D_FEAT, NUM_CLASSES), full(1, NUM_CLASSES),
        ],
        out_specs=[
            pl.BlockSpec((ROW_BLK, D_STRUCT_OUT), lambda i: (i, 0)),
            pl.BlockSpec((ROW_BLK, NUM_CLASSES), lambda i: (i, 0)),
        ],
        out_shape=[
            jax.ShapeDtypeStruct((N_NODES, D_STRUCT_OUT), jnp.float32),
            jax.ShapeDtypeStruct((N_NODES, NUM_CLASSES), jnp.float32),
        ],
    )(p0, p1, c0, c1, g0, g1, gi, cx,
      wc, bc.reshape(1, -1), wg2, bg2.reshape(1, -1), wl, bl.reshape(1, -1))


def kernel(client_x, structural_features, node_ids, edge_index,
           W_c, b_c, W_g1, b_g1, W_g2, b_g2, W_l, b_l):
    src = edge_index[0]
    dst = edge_index[1]
    epad = E_PAD - N_EDGES
    src_r = jnp.concatenate(
        [src, jnp.zeros((epad,), jnp.int32)]).reshape(NW, CH_PER_TILE, CHUNK)
    dst_r = jnp.concatenate(
        [dst, jnp.full((epad,), N_NODES, jnp.int32)]).reshape(NW, CH_PER_TILE, CHUNK)
    nid_r = jnp.concatenate(
        [node_ids, jnp.zeros((NID_PAD - N_NODES,), jnp.int32)]
    ).reshape(NW, NID_CH, CHUNK)
    zacc = jnp.zeros((ROWS_PER_TILE, D_FEAT), jnp.float32)
    zcnt = jnp.zeros((N_PAD,), jnp.float32)

    c0, c1 = _sc_count(dst_r, zcnt)
    c0 = c0.reshape(N_PAD, 1)
    c1 = c1.reshape(N_PAD, 1)
    p0, p1 = _sc_aggregate(structural_features, src_r, dst_r, zacc)
    s, iw = _tc_layer1(p0, p1, c0, c1, W_g1, b_g1)
    q0, q1 = _sc_aggregate(s, src_r, dst_r, zacc)
    g0, g1, gi = _sc_gather(q0, q1, iw, nid_r)
    S, out_client = _tc_final(
        q0, q1, c0, c1, g0, g1, gi, client_x,
        W_c, b_c, W_g2, b_g2, W_l, b_l)
    return (S, out_client)


# speedup vs baseline: 1.3780x; 1.0057x over previous
import jax
import jax.numpy as jnp
from jax import lax
from jax.experimental import pallas as pl
from jax.experimental.pallas import tpu as pltpu
from jax.experimental.pallas import tpu_sc as plsc

N_NODES = 10000
N_EDGES = 320000
D_FEAT = 128
D_HID = 128
D_STRUCT_OUT = 64
NUM_CLASSES = 40

NC = 2
NS = 16
NW = NC * NS

CHUNK = 128
CH_PER_TILE = -(-N_EDGES // (NW * CHUNK))
EDGES_PER_TILE = CH_PER_TILE * CHUNK
E_PAD = EDGES_PER_TILE * NW

N_PAD = N_NODES + 240
ROWS_PER_TILE = N_PAD // NS

NID_CH = -(-N_NODES // (NW * CHUNK))
NID_PAD = NID_CH * CHUNK * NW

ROW_BLK = 400
N_BLKS = N_NODES // ROW_BLK

_MESH = plsc.VectorSubcoreMesh(
    core_axis_name="c", subcore_axis_name="s", num_cores=NC, num_subcores=NS)


def _sc_agg_body(x, src_r, dst_r, zacc,
                 p0, p1,
                 idx_s_v, idx_d_v, rows_a, acc_sh, sem_a):
    c = lax.axis_index("c")
    s = lax.axis_index("s")
    t = c * NS + s
    row0 = pl.multiple_of(s * ROWS_PER_TILE, ROWS_PER_TILE)
    dsr = pl.ds(row0, ROWS_PER_TILE)

    pltpu.sync_copy(zacc, acc_sh.at[dsr])
    plsc.subcore_barrier()

    pltpu.sync_copy(src_r.at[t], idx_s_v)
    pltpu.sync_copy(dst_r.at[t], idx_d_v)

    def body(j, carry):
        pltpu.async_copy(x.at[idx_s_v.at[j]], rows_a, sem_a).wait()
        pltpu.sync_copy(rows_a, acc_sh.at[idx_d_v.at[j]], add=True)
        return carry

    lax.fori_loop(0, CH_PER_TILE, body, 0)

    plsc.subcore_barrier()

    @pl.when(c == 0)
    def _():
        pltpu.sync_copy(acc_sh.at[dsr], p0.at[dsr])

    @pl.when(c == 1)
    def _():
        pltpu.sync_copy(acc_sh.at[dsr], p1.at[dsr])


_sc_aggregate = pl.kernel(
    _sc_agg_body,
    out_type=(
        jax.ShapeDtypeStruct((N_PAD, D_FEAT), jnp.float32),
        jax.ShapeDtypeStruct((N_PAD, D_FEAT), jnp.float32),
    ),
    mesh=_MESH,
    scratch_types=[
        pltpu.VMEM((CH_PER_TILE, CHUNK), jnp.int32),
        pltpu.VMEM((CH_PER_TILE, CHUNK), jnp.int32),
        pltpu.VMEM((CHUNK, D_FEAT), jnp.float32),
        pltpu.VMEM_SHARED((N_PAD, D_FEAT), jnp.float32),
        pltpu.SemaphoreType.DMA,
    ],
)


def _sc_count_body(dst_r, zcnt,
                   c0, c1,
                   idx_d_v, cnt_v, red_v, res_v, cnt_sh):
    c = lax.axis_index("c")
    s = lax.axis_index("s")
    t = c * NS + s
    row0 = pl.multiple_of(s * ROWS_PER_TILE, ROWS_PER_TILE)
    dsr = pl.ds(row0, ROWS_PER_TILE)

    pltpu.sync_copy(dst_r.at[t], idx_d_v)
    pltpu.sync_copy(zcnt, cnt_v)
    ones16 = jnp.ones((16,), jnp.float32)

    def step(j, carry):
        def sub(k, carry2):
            idx = idx_d_v[j, pl.ds(k * 16, 16)]
            plsc.addupdate_scatter(cnt_v, [idx], ones16)
            return carry2
        return lax.fori_loop(0, CHUNK // 16, sub, carry)

    lax.fori_loop(0, CH_PER_TILE, step, 0)

    pltpu.sync_copy(cnt_v, cnt_sh.at[s])
    plsc.subcore_barrier()
    pltpu.sync_copy(cnt_sh.at[:, dsr], red_v)

    def red(r, carry):
        base = pl.multiple_of(r * 16, 16)
        acc = red_v[0, pl.ds(base, 16)]
        for q in range(1, NS):
            acc = acc + red_v[q, pl.ds(base, 16)]
        res_v[pl.ds(base, 16)] = acc
        return carry

    lax.fori_loop(0, ROWS_PER_TILE // 16, red, 0)

    @pl.when(c == 0)
    def _():
        pltpu.sync_copy(res_v, c0.at[dsr])

    @pl.when(c == 1)
    def _():
        pltpu.sync_copy(res_v, c1.at[dsr])


_sc_count = pl.kernel(
    _sc_count_body,
    out_type=(
        jax.ShapeDtypeStruct((N_PAD,), jnp.float32),
        jax.ShapeDtypeStruct((N_PAD,), jnp.float32),
    ),
    mesh=_MESH,
    compiler_params=pltpu.CompilerParams(needs_layout_passes=False),
    scratch_types=[
        pltpu.VMEM((CH_PER_TILE, CHUNK), jnp.int32),
        pltpu.VMEM((N_PAD,), jnp.float32),
        pltpu.VMEM((NS, ROWS_PER_TILE), jnp.float32),
        pltpu.VMEM((ROWS_PER_TILE,), jnp.float32),
        pltpu.VMEM_SHARED((NS, N_PAD), jnp.float32),
    ],
)


def _sc_gather_body(t0, t1, tw, nid_r,
                    g0, g1, gi,
                    nid_v, rows_v, sem):
    c = lax.axis_index("c")
    s = lax.axis_index("s")
    t = c * NS + s
    pltpu.sync_copy(nid_r.at[t], nid_v)

    def step(j, carry):
        base = pl.multiple_of(t * (NID_CH * CHUNK) + j * CHUNK, CHUNK)
        dso = pl.ds(base, CHUNK)
        pltpu.async_copy(t0.at[nid_v.at[j]], rows_v, sem).wait()
        pltpu.sync_copy(rows_v, g0.at[dso])
        pltpu.async_copy(t1.at[nid_v.at[j]], rows_v, sem).wait()
        pltpu.sync_copy(rows_v, g1.at[dso])
        pltpu.async_copy(tw.at[nid_v.at[j]], rows_v, sem).wait()
        pltpu.sync_copy(rows_v, gi.at[dso])
        return carry

    lax.fori_loop(0, NID_CH, step, 0)


_sc_gather = pl.kernel(
    _sc_gather_body,
    out_type=(
        jax.ShapeDtypeStruct((NID_PAD, D_FEAT), jnp.float32),
        jax.ShapeDtypeStruct((NID_PAD, D_FEAT), jnp.float32),
        jax.ShapeDtypeStruct((NID_PAD, D_FEAT), jnp.float32),
    ),
    mesh=_MESH,
    scratch_types=[
        pltpu.VMEM((NID_CH, CHUNK), jnp.int32),
        pltpu.VMEM((CHUNK, D_FEAT), jnp.float32),
        pltpu.SemaphoreType.DMA,
    ],
)


def _tc_layer1_body(p0, p1, c0, c1, w, b, s_out, iw_out):
    cnt = c0[...] + c1[...]
    inv = 1.0 / jnp.maximum(cnt, 1.0)
    agg = (p0[...] + p1[...]) * inv
    h = jnp.dot(agg, w[...], preferred_element_type=jnp.float32) + b[...]
    s_out[...] = jnp.maximum(h, 0.0)
    iw_out[...] = jnp.broadcast_to(inv, (ROW_BLK, D_FEAT))


def _tc_layer1(p0, p1, c0, c1, w, b):
    return pl.pallas_call(
        _tc_layer1_body,
        grid=(N_BLKS,),
        in_specs=[
            pl.BlockSpec((ROW_BLK, D_FEAT), lambda i: (i, 0)),
            pl.BlockSpec((ROW_BLK, D_FEAT), lambda i: (i, 0)),
            pl.BlockSpec((ROW_BLK, 1), lambda i: (i, 0)),
            pl.BlockSpec((ROW_BLK, 1), lambda i: (i, 0)),
            pl.BlockSpec((D_FEAT, D_HID), lambda i: (0, 0)),
            pl.BlockSpec((1, D_HID), lambda i: (0, 0)),
        ],
        out_specs=[
            pl.BlockSpec((ROW_BLK, D_HID), lambda i: (i, 0)),
            pl.BlockSpec((ROW_BLK, D_FEAT), lambda i: (i, 0)),
        ],
        out_shape=[
            jax.ShapeDtypeStruct((N_NODES, D_HID), jnp.float32),
            jax.ShapeDtypeStruct((N_NODES, D_FEAT), jnp.float32),
        ],
    )(p0, p1, c0, c1, w, b.reshape(1, D_HID))


def _tc_final_body(p0, p1, c0, c1, g0, g1, gi, cx,
                   wc, bc, wg2, bg2, wl, bl, s_out, cls_out):
    cnt = c0[...] + c1[...]
    inv = 1.0 / jnp.maximum(cnt, 1.0)
    agg = (p0[...] + p1[...]) * inv
    S = jnp.dot(agg, wg2[...], preferred_element_type=jnp.float32) + bg2[...]
    s_out[...] = S

    aggg = (g0[...] + g1[...]) * gi[:, 0:1]
    xs = jnp.dot(aggg, wg2[...], preferred_element_type=jnp.float32) + bg2[...]

    wl_top = wl[0:D_STRUCT_OUT, :]
    wl_bot = wl[D_STRUCT_OUT:, :]
    wcl = jnp.dot(wc[...], wl_top, preferred_element_type=jnp.float32)
    bfold = jnp.dot(bc[...], wl_top, preferred_element_type=jnp.float32) + bl[...]
    logits = (jnp.dot(cx[...], wcl, preferred_element_type=jnp.float32)
              + jnp.dot(xs, wl_bot, preferred_element_type=jnp.float32)
              + bfold)
    m = jnp.max(logits, axis=1, keepdims=True)
    e = jnp.exp(logits - m)
    cls_out[...] = e / jnp.sum(e, axis=1, keepdims=True)


def _tc_final(p0, p1, c0, c1, g0, g1, gi, cx, wc, bc, wg2, bg2, wl, bl):
    full = lambda r, k: pl.BlockSpec((r, k), lambda i: (0, 0))
    blk = lambda k: pl.BlockSpec((ROW_BLK, k), lambda i: (i, 0))
    return pl.pallas_call(
        _tc_final_body,
        grid=(N_BLKS,),
        in_specs=[
            blk(D_FEAT), blk(D_FEAT), blk(1), blk(1),
            blk(D_FEAT), blk(D_FEAT), blk(D_FEAT),
            blk(D_FEAT),
            full(D_FEAT, D_STRUCT_OUT), full(1, D_STRUCT_OUT),
            full(D_HID, D_STRUCT_OUT), full(1, D_STRUCT_OUT),
            full(D_FEAT, NUM_CLASSES), full(1, NUM_CLASSES),
        ],
        out_specs=[
            pl.BlockSpec((ROW_BLK, D_STRUCT_OUT), lambda i: (i, 0)),
            pl.BlockSpec((ROW_BLK, NUM_CLASSES), lambda i: (i, 0)),
        ],
        out_shape=[
            jax.ShapeDtypeStruct((N_NODES, D_STRUCT_OUT), jnp.float32),
            jax.ShapeDtypeStruct((N_NODES, NUM_CLASSES), jnp.float32),
        ],
    )(p0, p1, c0, c1, g0, g1, gi, cx,
      wc, bc.reshape(1, -1), wg2, bg2.reshape(1, -1), wl, bl.reshape(1, -1))


def kernel(client_x, structural_features, node_ids, edge_index,
           W_c, b_c, W_g1, b_g1, W_g2, b_g2, W_l, b_l):
    src = edge_index[0]
    dst = edge_index[1]
    epad = E_PAD - N_EDGES
    src_r = jnp.concatenate(
        [src, jnp.zeros((epad,), jnp.int32)]).reshape(NW, CH_PER_TILE, CHUNK)
    pad_dst = N_NODES + (jnp.arange(epad, dtype=jnp.int32) % (N_PAD - N_NODES))
    dst_r = jnp.concatenate(
        [dst, pad_dst]).reshape(NW, CH_PER_TILE, CHUNK)
    nid_r = jnp.concatenate(
        [node_ids, jnp.zeros((NID_PAD - N_NODES,), jnp.int32)]
    ).reshape(NW, NID_CH, CHUNK)
    zacc = jnp.zeros((ROWS_PER_TILE, D_FEAT), jnp.float32)
    zcnt = jnp.zeros((N_PAD,), jnp.float32)

    c0, c1 = _sc_count(dst_r, zcnt)
    c0 = c0.reshape(N_PAD, 1)
    c1 = c1.reshape(N_PAD, 1)
    p0, p1 = _sc_aggregate(structural_features, src_r, dst_r, zacc)
    s, iw = _tc_layer1(p0, p1, c0, c1, W_g1, b_g1)
    q0, q1 = _sc_aggregate(s, src_r, dst_r, zacc)
    g0, g1, gi = _sc_gather(q0, q1, iw, nid_r)
    S, out_client = _tc_final(
        q0, q1, c0, c1, g0, g1, gi, client_x,
        W_c, b_c, W_g2, b_g2, W_l, b_l)
    return (S, out_client)


# speedup vs baseline: 1.4389x; 1.0442x over previous
import jax
import jax.numpy as jnp
from jax import lax
from jax.experimental import pallas as pl
from jax.experimental.pallas import tpu as pltpu
from jax.experimental.pallas import tpu_sc as plsc

N_NODES = 10000
N_EDGES = 320000
D_FEAT = 128
D_HID = 128
D_STRUCT_OUT = 64
NUM_CLASSES = 40

NC = 2
NS = 16
NW = NC * NS

CHUNK = 128
CH_PER_TILE = -(-N_EDGES // (NW * CHUNK))
EDGES_PER_TILE = CH_PER_TILE * CHUNK
E_PAD = EDGES_PER_TILE * NW

N_PAD = N_NODES + 240
ROWS_PER_TILE = N_PAD // NS

NID_CH = -(-N_NODES // (NW * CHUNK))
NID_PAD = NID_CH * CHUNK * NW
NID5_CH = -(-N_NODES // (NS * CHUNK))
NID5_PAD = NID5_CH * CHUNK * NS

ROW_BLK = 400
N_BLKS = N_NODES // ROW_BLK

_MESH = plsc.VectorSubcoreMesh(
    core_axis_name="c", subcore_axis_name="s", num_cores=NC, num_subcores=NS)


def _sc_agg_body(x, src_r, dst_r, zacc,
                 p0, p1,
                 idx_s_v, idx_d_v, rows_a, acc_sh, sem_a):
    c = lax.axis_index("c")
    s = lax.axis_index("s")
    t = c * NS + s
    row0 = pl.multiple_of(s * ROWS_PER_TILE, ROWS_PER_TILE)
    dsr = pl.ds(row0, ROWS_PER_TILE)

    pltpu.sync_copy(zacc, acc_sh.at[dsr])
    plsc.subcore_barrier()

    pltpu.sync_copy(src_r.at[t], idx_s_v)
    pltpu.sync_copy(dst_r.at[t], idx_d_v)

    def body(j, carry):
        pltpu.async_copy(x.at[idx_s_v.at[j]], rows_a, sem_a).wait()
        pltpu.sync_copy(rows_a, acc_sh.at[idx_d_v.at[j]], add=True)
        return carry

    lax.fori_loop(0, CH_PER_TILE, body, 0)

    plsc.subcore_barrier()

    @pl.when(c == 0)
    def _():
        pltpu.sync_copy(acc_sh.at[dsr], p0.at[dsr])

    @pl.when(c == 1)
    def _():
        pltpu.sync_copy(acc_sh.at[dsr], p1.at[dsr])


_sc_aggregate = pl.kernel(
    _sc_agg_body,
    out_type=(
        jax.ShapeDtypeStruct((N_PAD, D_FEAT), jnp.float32),
        jax.ShapeDtypeStruct((N_PAD, D_FEAT), jnp.float32),
    ),
    mesh=_MESH,
    scratch_types=[
        pltpu.VMEM((CH_PER_TILE, CHUNK), jnp.int32),
        pltpu.VMEM((CH_PER_TILE, CHUNK), jnp.int32),
        pltpu.VMEM((CHUNK, D_FEAT), jnp.float32),
        pltpu.VMEM_SHARED((N_PAD, D_FEAT), jnp.float32),
        pltpu.SemaphoreType.DMA,
    ],
)


def _sc_count_body(dst_r, zcnt,
                   c0, c1,
                   idx_d_v, cnt_v, red_v, res_v, cnt_sh):
    c = lax.axis_index("c")
    s = lax.axis_index("s")
    t = c * NS + s
    row0 = pl.multiple_of(s * ROWS_PER_TILE, ROWS_PER_TILE)
    dsr = pl.ds(row0, ROWS_PER_TILE)

    pltpu.sync_copy(dst_r.at[t], idx_d_v)
    pltpu.sync_copy(zcnt, cnt_v)
    ones16 = jnp.ones((16,), jnp.float32)

    def step(j, carry):
        def sub(k, carry2):
            idx = idx_d_v[j, pl.ds(k * 16, 16)]
            plsc.addupdate_scatter(cnt_v, [idx], ones16)
            return carry2
        return lax.fori_loop(0, CHUNK // 16, sub, carry)

    lax.fori_loop(0, CH_PER_TILE, step, 0)

    pltpu.sync_copy(cnt_v, cnt_sh.at[s])
    plsc.subcore_barrier()
    pltpu.sync_copy(cnt_sh.at[:, dsr], red_v)

    def red(r, carry):
        base = pl.multiple_of(r * 16, 16)
        acc = red_v[0, pl.ds(base, 16)]
        for q in range(1, NS):
            acc = acc + red_v[q, pl.ds(base, 16)]
        res_v[pl.ds(base, 16)] = acc
        return carry

    lax.fori_loop(0, ROWS_PER_TILE // 16, red, 0)

    @pl.when(c == 0)
    def _():
        pltpu.sync_copy(res_v, c0.at[dsr])

    @pl.when(c == 1)
    def _():
        pltpu.sync_copy(res_v, c1.at[dsr])


_sc_count = pl.kernel(
    _sc_count_body,
    out_type=(
        jax.ShapeDtypeStruct((N_PAD,), jnp.float32),
        jax.ShapeDtypeStruct((N_PAD,), jnp.float32),
    ),
    mesh=_MESH,
    compiler_params=pltpu.CompilerParams(needs_layout_passes=False),
    scratch_types=[
        pltpu.VMEM((CH_PER_TILE, CHUNK), jnp.int32),
        pltpu.VMEM((N_PAD,), jnp.float32),
        pltpu.VMEM((NS, ROWS_PER_TILE), jnp.float32),
        pltpu.VMEM((ROWS_PER_TILE,), jnp.float32),
        pltpu.VMEM_SHARED((NS, N_PAD), jnp.float32),
    ],
)


def _sc_agg2_body(x, src_r, dst_r, zacc, iw, nid5, nid3,
                  q0, q1, g0, g1, gi,
                  idx_s_v, idx_d_v, rows_a, nid5_v, nid3_v, acc_sh, sem_a):
    c = lax.axis_index("c")
    s = lax.axis_index("s")
    t = c * NS + s
    row0 = pl.multiple_of(s * ROWS_PER_TILE, ROWS_PER_TILE)
    dsr = pl.ds(row0, ROWS_PER_TILE)

    pltpu.sync_copy(zacc, acc_sh.at[dsr])
    pltpu.sync_copy(src_r.at[t], idx_s_v)
    pltpu.sync_copy(dst_r.at[t], idx_d_v)
    pltpu.sync_copy(nid5.at[s], nid5_v)
    pltpu.sync_copy(nid3.at[t], nid3_v)
    plsc.subcore_barrier()

    def body(j, carry):
        pltpu.async_copy(x.at[idx_s_v.at[j]], rows_a, sem_a).wait()
        pltpu.sync_copy(rows_a, acc_sh.at[idx_d_v.at[j]], add=True)
        return carry

    lax.fori_loop(0, CH_PER_TILE, body, 0)
    plsc.subcore_barrier()

    @pl.when(c == 0)
    def _():
        pltpu.sync_copy(acc_sh.at[dsr], q0.at[dsr])

    @pl.when(c == 1)
    def _():
        pltpu.sync_copy(acc_sh.at[dsr], q1.at[dsr])

    plsc.subcore_barrier()

    for j in range(NID5_CH):
        base = pl.multiple_of(s * (NID5_CH * CHUNK) + j * CHUNK, CHUNK)
        dso = pl.ds(base, CHUNK)

        @pl.when(c == 0)
        def _():
            pltpu.async_copy(q0.at[nid5_v.at[j]], rows_a, sem_a).wait()
            pltpu.sync_copy(rows_a, g0.at[dso])

        @pl.when(c == 1)
        def _():
            pltpu.async_copy(q1.at[nid5_v.at[j]], rows_a, sem_a).wait()
            pltpu.sync_copy(rows_a, g1.at[dso])

    for j in range(NID_CH):
        base = pl.multiple_of(t * (NID_CH * CHUNK) + j * CHUNK, CHUNK)
        dso = pl.ds(base, CHUNK)
        pltpu.async_copy(iw.at[nid3_v.at[j]], rows_a, sem_a).wait()
        pltpu.sync_copy(rows_a, gi.at[dso])


_sc_agg2 = pl.kernel(
    _sc_agg2_body,
    out_type=(
        jax.ShapeDtypeStruct((N_PAD, D_FEAT), jnp.float32),
        jax.ShapeDtypeStruct((N_PAD, D_FEAT), jnp.float32),
        jax.ShapeDtypeStruct((NID5_PAD, D_FEAT), jnp.float32),
        jax.ShapeDtypeStruct((NID5_PAD, D_FEAT), jnp.float32),
        jax.ShapeDtypeStruct((NID_PAD, D_FEAT), jnp.float32),
    ),
    mesh=_MESH,
    scratch_types=[
        pltpu.VMEM((CH_PER_TILE, CHUNK), jnp.int32),
        pltpu.VMEM((CH_PER_TILE, CHUNK), jnp.int32),
        pltpu.VMEM((CHUNK, D_FEAT), jnp.float32),
        pltpu.VMEM((NID5_CH, CHUNK), jnp.int32),
        pltpu.VMEM((NID_CH, CHUNK), jnp.int32),
        pltpu.VMEM_SHARED((N_PAD, D_FEAT), jnp.float32),
        pltpu.SemaphoreType.DMA,
    ],
)


def _sc_gather_body(t0, t1, tw, nid_r,
                    g0, g1, gi,
                    nid_v, rows_v, sem):
    c = lax.axis_index("c")
    s = lax.axis_index("s")
    t = c * NS + s
    pltpu.sync_copy(nid_r.at[t], nid_v)

    def step(j, carry):
        base = pl.multiple_of(t * (NID_CH * CHUNK) + j * CHUNK, CHUNK)
        dso = pl.ds(base, CHUNK)
        pltpu.async_copy(t0.at[nid_v.at[j]], rows_v, sem).wait()
        pltpu.sync_copy(rows_v, g0.at[dso])
        pltpu.async_copy(t1.at[nid_v.at[j]], rows_v, sem).wait()
        pltpu.sync_copy(rows_v, g1.at[dso])
        pltpu.async_copy(tw.at[nid_v.at[j]], rows_v, sem).wait()
        pltpu.sync_copy(rows_v, gi.at[dso])
        return carry

    lax.fori_loop(0, NID_CH, step, 0)


_sc_gather = pl.kernel(
    _sc_gather_body,
    out_type=(
        jax.ShapeDtypeStruct((NID_PAD, D_FEAT), jnp.float32),
        jax.ShapeDtypeStruct((NID_PAD, D_FEAT), jnp.float32),
        jax.ShapeDtypeStruct((NID_PAD, D_FEAT), jnp.float32),
    ),
    mesh=_MESH,
    scratch_types=[
        pltpu.VMEM((NID_CH, CHUNK), jnp.int32),
        pltpu.VMEM((CHUNK, D_FEAT), jnp.float32),
        pltpu.SemaphoreType.DMA,
    ],
)


def _tc_layer1_body(p0, p1, c0, c1, w, b, s_out, iw_out):
    cnt = c0[...] + c1[...]
    inv = 1.0 / jnp.maximum(cnt, 1.0)
    agg = (p0[...] + p1[...]) * inv
    h = jnp.dot(agg, w[...], preferred_element_type=jnp.float32) + b[...]
    s_out[...] = jnp.maximum(h, 0.0)
    iw_out[...] = jnp.broadcast_to(inv, (ROW_BLK, D_FEAT))


def _tc_layer1(p0, p1, c0, c1, w, b):
    return pl.pallas_call(
        _tc_layer1_body,
        grid=(N_BLKS,),
        in_specs=[
            pl.BlockSpec((ROW_BLK, D_FEAT), lambda i: (i, 0)),
            pl.BlockSpec((ROW_BLK, D_FEAT), lambda i: (i, 0)),
            pl.BlockSpec((ROW_BLK, 1), lambda i: (i, 0)),
            pl.BlockSpec((ROW_BLK, 1), lambda i: (i, 0)),
            pl.BlockSpec((D_FEAT, D_HID), lambda i: (0, 0)),
            pl.BlockSpec((1, D_HID), lambda i: (0, 0)),
        ],
        out_specs=[
            pl.BlockSpec((ROW_BLK, D_HID), lambda i: (i, 0)),
            pl.BlockSpec((ROW_BLK, D_FEAT), lambda i: (i, 0)),
        ],
        out_shape=[
            jax.ShapeDtypeStruct((N_NODES, D_HID), jnp.float32),
            jax.ShapeDtypeStruct((N_NODES, D_FEAT), jnp.float32),
        ],
    )(p0, p1, c0, c1, w, b.reshape(1, D_HID))


def _tc_final_body(p0, p1, c0, c1, g0, g1, gi, cx,
                   wc, bc, wg2, bg2, wl, bl, s_out, cls_out):
    cnt = c0[...] + c1[...]
    inv = 1.0 / jnp.maximum(cnt, 1.0)
    agg = (p0[...] + p1[...]) * inv
    S = jnp.dot(agg, wg2[...], preferred_element_type=jnp.float32) + bg2[...]
    s_out[...] = S

    aggg = (g0[...] + g1[...]) * gi[:, 0:1]
    xs = jnp.dot(aggg, wg2[...], preferred_element_type=jnp.float32) + bg2[...]

    wl_top = wl[0:D_STRUCT_OUT, :]
    wl_bot = wl[D_STRUCT_OUT:, :]
    wcl = jnp.dot(wc[...], wl_top, preferred_element_type=jnp.float32)
    bfold = jnp.dot(bc[...], wl_top, preferred_element_type=jnp.float32) + bl[...]
    logits = (jnp.dot(cx[...], wcl, preferred_element_type=jnp.float32)
              + jnp.dot(xs, wl_bot, preferred_element_type=jnp.float32)
              + bfold)
    m = jnp.max(logits, axis=1, keepdims=True)
    e = jnp.exp(logits - m)
    cls_out[...] = e / jnp.sum(e, axis=1, keepdims=True)


def _tc_final(p0, p1, c0, c1, g0, g1, gi, cx, wc, bc, wg2, bg2, wl, bl):
    full = lambda r, k: pl.BlockSpec((r, k), lambda i: (0, 0))
    blk = lambda k: pl.BlockSpec((ROW_BLK, k), lambda i: (i, 0))
    return pl.pallas_call(
        _tc_final_body,
        grid=(N_BLKS,),
        in_specs=[
            blk(D_FEAT), blk(D_FEAT), blk(1), blk(1),
            blk(D_FEAT), blk(D_FEAT), blk(D_FEAT),
            blk(D_FEAT),
            full(D_FEAT, D_STRUCT_OUT), full(1, D_STRUCT_OUT),
            full(D_HID, D_STRUCT_OUT), full(1, D_STRUCT_OUT),
            full(D_FEAT, NUM_CLASSES), full(1, NUM_CLASSES),
        ],
        out_specs=[
            pl.BlockSpec((ROW_BLK, D_STRUCT_OUT), lambda i: (i, 0)),
            pl.BlockSpec((ROW_BLK, NUM_CLASSES), lambda i: (i, 0)),
        ],
        out_shape=[
            jax.ShapeDtypeStruct((N_NODES, D_STRUCT_OUT), jnp.float32),
            jax.ShapeDtypeStruct((N_NODES, NUM_CLASSES), jnp.float32),
        ],
    )(p0, p1, c0, c1, g0, g1, gi, cx,
      wc, bc.reshape(1, -1), wg2, bg2.reshape(1, -1), wl, bl.reshape(1, -1))


def kernel(client_x, structural_features, node_ids, edge_index,
           W_c, b_c, W_g1, b_g1, W_g2, b_g2, W_l, b_l):
    src = edge_index[0]
    dst = edge_index[1]
    epad = E_PAD - N_EDGES
    src_r = jnp.concatenate(
        [src, jnp.zeros((epad,), jnp.int32)]).reshape(NW, CH_PER_TILE, CHUNK)
    pad_dst = N_NODES + (jnp.arange(epad, dtype=jnp.int32) % (N_PAD - N_NODES))
    dst_r = jnp.concatenate(
        [dst, pad_dst]).reshape(NW, CH_PER_TILE, CHUNK)
    nid3_r = jnp.concatenate(
        [node_ids, jnp.zeros((NID_PAD - N_NODES,), jnp.int32)]
    ).reshape(NW, NID_CH, CHUNK)
    nid5_r = jnp.concatenate(
        [node_ids, jnp.zeros((NID5_PAD - N_NODES,), jnp.int32)]
    ).reshape(NS, NID5_CH, CHUNK)
    zacc = jnp.zeros((ROWS_PER_TILE, D_FEAT), jnp.float32)
    zcnt = jnp.zeros((N_PAD,), jnp.float32)

    c0, c1 = _sc_count(dst_r, zcnt)
    c0 = c0.reshape(N_PAD, 1)
    c1 = c1.reshape(N_PAD, 1)
    p0, p1 = _sc_aggregate(structural_features, src_r, dst_r, zacc)
    s, iw = _tc_layer1(p0, p1, c0, c1, W_g1, b_g1)
    q0, q1, g0, g1, gi = _sc_agg2(s, src_r, dst_r, zacc, iw, nid5_r, nid3_r)
    S, out_client = _tc_final(
        q0, q1, c0, c1, g0, g1, gi, client_x,
        W_c, b_c, W_g2, b_g2, W_l, b_l)
    return (S, out_client)
